# baseline (speedup 1.0000x reference)
import functools
import math

import jax
import jax.numpy as jnp
from jax import lax
from jax.experimental import pallas as pl
from jax.experimental.pallas import tpu as pltpu
from jax.experimental.pallas import tpu_sc as plsc

VOCAB = 1000000
EMBED_DIM = 64
HARMONIC_BASES = 16

_B, _S = 1024, 50
_T = _B * _S
_NC, _NS = 2, 16
_NW = _NC * _NS
_TPW = _T // _NW
_EPW = _TPW * HARMONIC_BASES
_PR = _T // 8


def _sc_gather(ilist, flat_a, flat_p):
    mesh = plsc.VectorSubcoreMesh(core_axis_name="c", subcore_axis_name="s")

    @functools.partial(
        pl.kernel,
        out_type=(
            jax.ShapeDtypeStruct((_T * HARMONIC_BASES,), jnp.float32),
            jax.ShapeDtypeStruct((_T * HARMONIC_BASES,), jnp.float32),
        ),
        name="sc_spectral_gather",
        mesh=mesh,
        scratch_types=[
            pltpu.VMEM((_EPW,), jnp.int32),
            pltpu.VMEM((_EPW,), jnp.float32),
            pltpu.VMEM((_EPW,), jnp.float32),
            pltpu.SemaphoreType.DMA,
        ],
        compiler_params=pltpu.CompilerParams(use_tc_tiling_on_sc=False),
    )
    def gather_kernel(ilist_hbm, a_hbm, p_hbm, a_out, p_out,
                      ilist_v, vals_a, vals_p, sem):
        wid = lax.axis_index("s") * _NC + lax.axis_index("c")
        base = wid * _EPW
        pltpu.sync_copy(ilist_hbm.at[pl.ds(base, _EPW)], ilist_v)
        cp_a = pltpu.async_copy(a_hbm.at[ilist_v], vals_a, sem)
        cp_p = pltpu.async_copy(p_hbm.at[ilist_v], vals_p, sem)
        cp_a.wait()
        cp_p.wait()
        pltpu.sync_copy(vals_a, a_out.at[pl.ds(base, _EPW)])
        pltpu.sync_copy(vals_p, p_out.at[pl.ds(base, _EPW)])

    return gather_kernel(ilist, flat_a, flat_p)


_BR = 320


def _tc_body(amp_ref, phase_ref, sb_ref, cb_ref, out_ref):
    a = amp_ref[...]
    p = phase_ref[...]
    w = a * jnp.cos(p)
    z = a * jnp.sin(p)
    out_ref[...] = (
        jnp.dot(w, sb_ref[...], preferred_element_type=jnp.float32)
        + jnp.dot(z, cb_ref[...], preferred_element_type=jnp.float32)
    )


def _tc_synth(amp_p, phase_p, sb, cb):
    grid = (_PR // _BR,)
    return pl.pallas_call(
        _tc_body,
        grid=grid,
        in_specs=[
            pl.BlockSpec((_BR, 128), lambda i: (i, 0)),
            pl.BlockSpec((_BR, 128), lambda i: (i, 0)),
            pl.BlockSpec((128, 8 * EMBED_DIM), lambda i: (0, 0)),
            pl.BlockSpec((128, 8 * EMBED_DIM), lambda i: (0, 0)),
        ],
        out_specs=pl.BlockSpec((_BR, 8 * EMBED_DIM), lambda i: (i, 0)),
        out_shape=jax.ShapeDtypeStruct((_PR, 8 * EMBED_DIM), jnp.float32),
    )(amp_p, phase_p, sb, cb)


_W = 1664
_NB = 601
_NR = HARMONIC_BASES * _W // 128
_FLAT = _NB * _NR * 128


def _detile_body(in_ref, out_ref):
    out_ref[...] = in_ref[...].reshape(_NR, 128)


def _flatten_table(tab):
    q2 = pl.pallas_call(
        _detile_body,
        grid=(_NB,),
        in_specs=[pl.BlockSpec((HARMONIC_BASES, _W), lambda j: (0, j))],
        out_specs=pl.BlockSpec((_NR, 128), lambda j: (j, 0)),
        out_shape=jax.ShapeDtypeStruct((_NB * _NR, 128), jnp.float32),
    )(tab.T)
    return q2.reshape(_FLAT)


def kernel(x, frequency_amplitudes, frequency_phases, frequencies):
    idx = x.reshape(_T).astype(jnp.int32)
    base = idx + (idx // _W) * ((HARMONIC_BASES - 1) * _W)
    harm = jnp.tile(jnp.arange(HARMONIC_BASES, dtype=jnp.int32) * _W, _T)
    ilist = jnp.repeat(base, HARMONIC_BASES) + harm
    flat_a = _flatten_table(frequency_amplitudes)
    flat_p = _flatten_table(frequency_phases)

    a_flat, p_flat = _sc_gather(ilist, flat_a, flat_p)
    amp_p = a_flat.reshape(_PR, 128)
    phase_p = p_flat.reshape(_PR, 128)

    t = jnp.linspace(0.0, 1.0, EMBED_DIM, dtype=jnp.float32)
    theta = (2.0 * math.pi) * frequencies[:, None] * t[None, :]
    eye8 = jnp.eye(8, dtype=jnp.float32)
    sb = jnp.kron(eye8, jnp.sin(theta))
    cb = jnp.kron(eye8, jnp.cos(theta))

    out = _tc_synth(amp_p, phase_p, sb, cb)
    return out.reshape(_B, _S, EMBED_DIM)


# speedup vs baseline: 3.1871x; 3.1871x over previous
import functools
import math

import jax
import jax.numpy as jnp
from jax import lax
from jax.experimental import pallas as pl
from jax.experimental.pallas import tpu as pltpu
from jax.experimental.pallas import tpu_sc as plsc

VOCAB = 1000000
EMBED_DIM = 64
HARMONIC_BASES = 16

_B, _S = 1024, 50
_T = _B * _S
_NC, _NS = 2, 16
_NW = _NC * _NS
_TPW = _T // _NW
_EPW = _TPW * HARMONIC_BASES
_PR = _T // 8


def _sc_gather(ilist, flat_a, flat_p):
    mesh = plsc.VectorSubcoreMesh(core_axis_name="c", subcore_axis_name="s")

    @functools.partial(
        pl.kernel,
        out_type=(
            jax.ShapeDtypeStruct((_T * HARMONIC_BASES,), jnp.float32),
            jax.ShapeDtypeStruct((_T * HARMONIC_BASES,), jnp.float32),
        ),
        name="sc_spectral_gather",
        mesh=mesh,
        scratch_types=[
            pltpu.VMEM((_EPW,), jnp.int32),
            pltpu.VMEM((_EPW,), jnp.float32),
            pltpu.VMEM((_EPW,), jnp.float32),
            pltpu.SemaphoreType.DMA,
        ],
        compiler_params=pltpu.CompilerParams(use_tc_tiling_on_sc=False),
    )
    def gather_kernel(ilist_hbm, a_hbm, p_hbm, a_out, p_out,
                      ilist_v, vals_a, vals_p, sem):
        wid = lax.axis_index("s") * _NC + lax.axis_index("c")
        base = wid * _EPW
        pltpu.sync_copy(ilist_hbm.at[pl.ds(base, _EPW)], ilist_v)
        cp_a = pltpu.async_copy(a_hbm.at[ilist_v], vals_a, sem)
        cp_p = pltpu.async_copy(p_hbm.at[ilist_v], vals_p, sem)
        cp_a.wait()
        cp_p.wait()
        pltpu.sync_copy(vals_a, a_out.at[pl.ds(base, _EPW)])
        pltpu.sync_copy(vals_p, p_out.at[pl.ds(base, _EPW)])

    return gather_kernel(ilist, flat_a, flat_p)


_BR = 320


def _tc_body(amp_ref, phase_ref, sb_ref, cb_ref, out_ref):
    a = amp_ref[...]
    p = phase_ref[...]
    w = a * jnp.cos(p)
    z = a * jnp.sin(p)
    out_ref[...] = (
        jnp.dot(w, sb_ref[...], preferred_element_type=jnp.float32)
        + jnp.dot(z, cb_ref[...], preferred_element_type=jnp.float32)
    )


def _tc_synth(amp_p, phase_p, sb, cb):
    grid = (_PR // _BR,)
    return pl.pallas_call(
        _tc_body,
        grid=grid,
        in_specs=[
            pl.BlockSpec((_BR, 128), lambda i: (i, 0)),
            pl.BlockSpec((_BR, 128), lambda i: (i, 0)),
            pl.BlockSpec((128, 8 * EMBED_DIM), lambda i: (0, 0)),
            pl.BlockSpec((128, 8 * EMBED_DIM), lambda i: (0, 0)),
        ],
        out_specs=pl.BlockSpec((_BR, 8 * EMBED_DIM), lambda i: (i, 0)),
        out_shape=jax.ShapeDtypeStruct((_PR, 8 * EMBED_DIM), jnp.float32),
    )(amp_p, phase_p, sb, cb)


_W = 76928
_NB = 13
_NR = HARMONIC_BASES * _W // 128
_FLAT = _NB * _NR * 128


def _detile_body(in_ref, out_ref):
    out_ref[...] = in_ref[...].reshape(_NR, 128)


def _flatten_table(tab):
    q2 = pl.pallas_call(
        _detile_body,
        grid=(_NB,),
        in_specs=[pl.BlockSpec((HARMONIC_BASES, _W), lambda j: (0, j))],
        out_specs=pl.BlockSpec((_NR, 128), lambda j: (j, 0)),
        out_shape=jax.ShapeDtypeStruct((_NB * _NR, 128), jnp.float32),
    )(tab.T)
    return q2.reshape(_FLAT)


def kernel(x, frequency_amplitudes, frequency_phases, frequencies):
    idx = x.reshape(_T).astype(jnp.int32)
    base = idx + (idx // _W) * ((HARMONIC_BASES - 1) * _W)
    harm = jnp.tile(jnp.arange(HARMONIC_BASES, dtype=jnp.int32) * _W, _T)
    ilist = jnp.repeat(base, HARMONIC_BASES) + harm
    flat_a = _flatten_table(frequency_amplitudes)
    flat_p = _flatten_table(frequency_phases)

    a_flat, p_flat = _sc_gather(ilist, flat_a, flat_p)
    amp_p = a_flat.reshape(_PR, 128)
    phase_p = p_flat.reshape(_PR, 128)

    t = jnp.linspace(0.0, 1.0, EMBED_DIM, dtype=jnp.float32)
    theta = (2.0 * math.pi) * frequencies[:, None] * t[None, :]
    eye8 = jnp.eye(8, dtype=jnp.float32)
    sb = jnp.kron(eye8, jnp.sin(theta))
    cb = jnp.kron(eye8, jnp.cos(theta))

    out = _tc_synth(amp_p, phase_p, sb, cb)
    return out.reshape(_B, _S, EMBED_DIM)


# speedup vs baseline: 3.2595x; 1.0227x over previous
import functools
import math

import jax
import jax.numpy as jnp
from jax import lax
from jax.experimental import pallas as pl
from jax.experimental.pallas import tpu as pltpu
from jax.experimental.pallas import tpu_sc as plsc

VOCAB = 1000000
EMBED_DIM = 64
HARMONIC_BASES = 16

_B, _S = 1024, 50
_T = _B * _S
_NC, _NS = 2, 16
_NW = _NC * _NS
_TPW = _T // _NW
_EPW = _TPW * HARMONIC_BASES
_PR = _T // 8


def _sc_gather_one(ilist, flat_tab, name):
    mesh = plsc.VectorSubcoreMesh(core_axis_name="c", subcore_axis_name="s")

    @functools.partial(
        pl.kernel,
        out_type=jax.ShapeDtypeStruct((_T * HARMONIC_BASES,), jnp.float32),
        name=name,
        mesh=mesh,
        scratch_types=[
            pltpu.VMEM((_EPW,), jnp.int32),
            pltpu.VMEM((_EPW,), jnp.float32),
            pltpu.SemaphoreType.DMA,
        ],
        compiler_params=pltpu.CompilerParams(use_tc_tiling_on_sc=False),
    )
    def gather_kernel(ilist_hbm, tab_hbm, out_hbm, ilist_v, vals, sem):
        wid = lax.axis_index("s") * _NC + lax.axis_index("c")
        base = wid * _EPW
        pltpu.sync_copy(ilist_hbm.at[pl.ds(base, _EPW)], ilist_v)
        pltpu.async_copy(tab_hbm.at[ilist_v], vals, sem).wait()
        pltpu.sync_copy(vals, out_hbm.at[pl.ds(base, _EPW)])

    return gather_kernel(ilist, flat_tab)


_BR = 320


def _tc_body(amp_ref, phase_ref, sb_ref, cb_ref, out_ref):
    a = amp_ref[...]
    p = phase_ref[...]
    w = a * jnp.cos(p)
    z = a * jnp.sin(p)
    out_ref[...] = (
        jnp.dot(w, sb_ref[...], preferred_element_type=jnp.float32)
        + jnp.dot(z, cb_ref[...], preferred_element_type=jnp.float32)
    )


def _tc_synth(amp_p, phase_p, sb, cb):
    grid = (_PR // _BR,)
    return pl.pallas_call(
        _tc_body,
        grid=grid,
        in_specs=[
            pl.BlockSpec((_BR, 128), lambda i: (i, 0)),
            pl.BlockSpec((_BR, 128), lambda i: (i, 0)),
            pl.BlockSpec((128, 8 * EMBED_DIM), lambda i: (0, 0)),
            pl.BlockSpec((128, 8 * EMBED_DIM), lambda i: (0, 0)),
        ],
        out_specs=pl.BlockSpec((_BR, 8 * EMBED_DIM), lambda i: (i, 0)),
        out_shape=jax.ShapeDtypeStruct((_PR, 8 * EMBED_DIM), jnp.float32),
    )(amp_p, phase_p, sb, cb)


_W = 76928
_NB = 13
_NR = HARMONIC_BASES * _W // 128
_FLAT = _NB * _NR * 128


def _detile_body(in_ref, out_ref):
    out_ref[...] = in_ref[...].reshape(_NR, 128)


def _flatten_table(tab):
    q2 = pl.pallas_call(
        _detile_body,
        grid=(_NB,),
        in_specs=[pl.BlockSpec((HARMONIC_BASES, _W), lambda j: (0, j))],
        out_specs=pl.BlockSpec((_NR, 128), lambda j: (j, 0)),
        out_shape=jax.ShapeDtypeStruct((_NB * _NR, 128), jnp.float32),
    )(tab.T)
    return q2.reshape(_FLAT)


def kernel(x, frequency_amplitudes, frequency_phases, frequencies):
    idx = x.reshape(_T).astype(jnp.int32)
    base = idx + (idx // _W) * ((HARMONIC_BASES - 1) * _W)
    harm = jnp.tile(jnp.arange(HARMONIC_BASES, dtype=jnp.int32) * _W, _T)
    ilist = jnp.repeat(base, HARMONIC_BASES) + harm
    flat_a = _flatten_table(frequency_amplitudes)
    a_flat = _sc_gather_one(ilist, flat_a, "sc_gather_amp")
    flat_p = _flatten_table(frequency_phases)
    p_flat = _sc_gather_one(ilist, flat_p, "sc_gather_phase")
    amp_p = a_flat.reshape(_PR, 128)
    phase_p = p_flat.reshape(_PR, 128)

    t = jnp.linspace(0.0, 1.0, EMBED_DIM, dtype=jnp.float32)
    theta = (2.0 * math.pi) * frequencies[:, None] * t[None, :]
    eye8 = jnp.eye(8, dtype=jnp.float32)
    sb = jnp.kron(eye8, jnp.sin(theta))
    cb = jnp.kron(eye8, jnp.cos(theta))

    out = _tc_synth(amp_p, phase_p, sb, cb)
    return out.reshape(_B, _S, EMBED_DIM)


# speedup vs baseline: 3.6362x; 1.1156x over previous
import functools
import math

import jax
import jax.numpy as jnp
from jax import lax
from jax.experimental import pallas as pl
from jax.experimental.pallas import tpu as pltpu
from jax.experimental.pallas import tpu_sc as plsc

VOCAB = 1000000
EMBED_DIM = 64
HARMONIC_BASES = 16

_B, _S = 1024, 50
_T = _B * _S
_NC, _NS = 2, 16
_NW = _NC * _NS
_TPW = _T // _NW
_EPW = _TPW * HARMONIC_BASES
_PR = _T // 8


def _sc_gather_one(idx, flat_tab, name):
    mesh = plsc.VectorSubcoreMesh(core_axis_name="c", subcore_axis_name="s")

    @functools.partial(
        pl.kernel,
        out_type=jax.ShapeDtypeStruct((_T * HARMONIC_BASES,), jnp.float32),
        name=name,
        mesh=mesh,
        scratch_types=[
            pltpu.VMEM((_TPW,), jnp.int32),
            pltpu.VMEM((_EPW,), jnp.int32),
            pltpu.VMEM((_EPW,), jnp.float32),
            pltpu.SemaphoreType.DMA,
        ],
        compiler_params=pltpu.CompilerParams(use_tc_tiling_on_sc=False),
    )
    def gather_kernel(idx_hbm, tab_hbm, out_hbm, idx_v, ilist_v, vals, sem):
        wid = lax.axis_index("s") * _NC + lax.axis_index("c")
        base = wid * _TPW
        pltpu.sync_copy(idx_hbm.at[pl.ds(base, _TPW)], idx_v)
        harm = lax.iota(jnp.int32, 16) * _W

        def build(k, carry):
            base16 = idx_v[pl.ds(k * 16, 16)]
            for j in range(16):
                bj = base16[jnp.full((16,), j, jnp.int32)]
                ilist_v[pl.ds((k * 16 + j) * 16, 16)] = bj + harm
            return carry

        lax.fori_loop(0, _TPW // 16, build, 0)
        pltpu.async_copy(tab_hbm.at[ilist_v], vals, sem).wait()
        pltpu.sync_copy(vals, out_hbm.at[pl.ds(wid * _EPW, _EPW)])

    return gather_kernel(idx, flat_tab)


_BR = 320


def _tc_body(amp_ref, phase_ref, sb_ref, cb_ref, out_ref):
    a = amp_ref[...]
    p = phase_ref[...]
    w = a * jnp.cos(p)
    z = a * jnp.sin(p)
    out_ref[...] = (
        jnp.dot(w, sb_ref[...], preferred_element_type=jnp.float32)
        + jnp.dot(z, cb_ref[...], preferred_element_type=jnp.float32)
    )


def _tc_synth(amp_p, phase_p, sb, cb):
    grid = (_PR // _BR,)
    return pl.pallas_call(
        _tc_body,
        grid=grid,
        in_specs=[
            pl.BlockSpec((_BR, 128), lambda i: (i, 0)),
            pl.BlockSpec((_BR, 128), lambda i: (i, 0)),
            pl.BlockSpec((128, 8 * EMBED_DIM), lambda i: (0, 0)),
            pl.BlockSpec((128, 8 * EMBED_DIM), lambda i: (0, 0)),
        ],
        out_specs=pl.BlockSpec((_BR, 8 * EMBED_DIM), lambda i: (i, 0)),
        out_shape=jax.ShapeDtypeStruct((_PR, 8 * EMBED_DIM), jnp.float32),
    )(amp_p, phase_p, sb, cb)


_W = 76928
_NB = 13
_NR = HARMONIC_BASES * _W // 128
_FLAT = _NB * _NR * 128


def _detile_body(in_ref, out_ref):
    out_ref[...] = in_ref[...].reshape(_NR, 128)


def _flatten_table(tab):
    q2 = pl.pallas_call(
        _detile_body,
        grid=(_NB,),
        in_specs=[pl.BlockSpec((HARMONIC_BASES, _W), lambda j: (0, j))],
        out_specs=pl.BlockSpec((_NR, 128), lambda j: (j, 0)),
        out_shape=jax.ShapeDtypeStruct((_NB * _NR, 128), jnp.float32),
    )(tab.T)
    return q2.reshape(_FLAT)


def kernel(x, frequency_amplitudes, frequency_phases, frequencies):
    idx = x.reshape(_T).astype(jnp.int32)
    base = idx + (idx // _W) * ((HARMONIC_BASES - 1) * _W)
    flat_a = _flatten_table(frequency_amplitudes)
    a_flat = _sc_gather_one(base, flat_a, "sc_gather_amp")
    flat_p = _flatten_table(frequency_phases)
    p_flat = _sc_gather_one(base, flat_p, "sc_gather_phase")
    amp_p = a_flat.reshape(_PR, 128)
    phase_p = p_flat.reshape(_PR, 128)

    t = jnp.linspace(0.0, 1.0, EMBED_DIM, dtype=jnp.float32)
    theta = (2.0 * math.pi) * frequencies[:, None] * t[None, :]
    eye8 = jnp.eye(8, dtype=jnp.float32)
    sb = jnp.kron(eye8, jnp.sin(theta))
    cb = jnp.kron(eye8, jnp.cos(theta))

    out = _tc_synth(amp_p, phase_p, sb, cb)
    return out.reshape(_B, _S, EMBED_DIM)


# speedup vs baseline: 4.5563x; 1.2530x over previous
import functools
import math

import jax
import jax.numpy as jnp
from jax import lax
from jax.experimental import pallas as pl
from jax.experimental.pallas import tpu as pltpu
from jax.experimental.pallas import tpu_sc as plsc

VOCAB = 1000000
EMBED_DIM = 64
HARMONIC_BASES = 16

_B, _S = 1024, 50
_T = _B * _S
_NC, _NS = 2, 16
_NW = _NC * _NS
_TPW = _T // _NW
_EPW = _TPW * HARMONIC_BASES
_PR = _T // 8

_W = 76928
_NB = 13
_NR = HARMONIC_BASES * _W // 128
_FLAT = _NB * _NR * 128


def _detile_body(a_ref, p_ref, out_ref):
    a = a_ref[...].reshape(_NR, 128).astype(jnp.bfloat16)
    p = p_ref[...].reshape(_NR, 128).astype(jnp.bfloat16)
    a32 = lax.bitcast_convert_type(a, jnp.uint16).astype(jnp.int32)
    p32 = lax.bitcast_convert_type(p, jnp.uint16).astype(jnp.int32)
    out_ref[...] = a32 | (p32 << 16)


def _flatten_pair(tab_a, tab_p):
    q2 = pl.pallas_call(
        _detile_body,
        grid=(_NB,),
        in_specs=[
            pl.BlockSpec((HARMONIC_BASES, _W), lambda j: (0, j)),
            pl.BlockSpec((HARMONIC_BASES, _W), lambda j: (0, j)),
        ],
        out_specs=pl.BlockSpec((_NR, 128), lambda j: (j, 0)),
        out_shape=jax.ShapeDtypeStruct((_NB * _NR, 128), jnp.int32),
    )(tab_a.T, tab_p.T)
    return q2.reshape(_FLAT)


def _sc_gather(base, flat_ap):
    mesh = plsc.VectorSubcoreMesh(core_axis_name="c", subcore_axis_name="s")

    @functools.partial(
        pl.kernel,
        out_type=jax.ShapeDtypeStruct((_T * HARMONIC_BASES,), jnp.int32),
        name="sc_spectral_gather",
        mesh=mesh,
        scratch_types=[
            pltpu.VMEM((_TPW,), jnp.int32),
            pltpu.VMEM((_EPW,), jnp.int32),
            pltpu.VMEM((_EPW,), jnp.int32),
            pltpu.SemaphoreType.DMA,
        ],
        compiler_params=pltpu.CompilerParams(use_tc_tiling_on_sc=False),
    )
    def gather_kernel(base_hbm, tab_hbm, out_hbm, base_v, ilist_v, vals, sem):
        wid = lax.axis_index("s") * _NC + lax.axis_index("c")
        tok0 = wid * _TPW
        pltpu.sync_copy(base_hbm.at[pl.ds(tok0, _TPW)], base_v)
        harm = lax.iota(jnp.int32, 16) * _W

        def build(k, carry):
            base16 = base_v[pl.ds(k * 16, 16)]
            for j in range(16):
                bj = base16[jnp.full((16,), j, jnp.int32)]
                ilist_v[pl.ds((k * 16 + j) * 16, 16)] = bj + harm
            return carry

        lax.fori_loop(0, _TPW // 16, build, 0)
        pltpu.async_copy(tab_hbm.at[ilist_v], vals, sem).wait()
        pltpu.sync_copy(vals, out_hbm.at[pl.ds(wid * _EPW, _EPW)])

    return gather_kernel(base, flat_ap)


_BR = 320


def _tc_body(ap_ref, sb_ref, cb_ref, out_ref):
    u = ap_ref[...]
    a = lax.bitcast_convert_type(
        (u & 0xFFFF).astype(jnp.uint16), jnp.bfloat16).astype(jnp.float32)
    p = lax.bitcast_convert_type(
        lax.shift_right_logical(u, 16).astype(jnp.uint16),
        jnp.bfloat16).astype(jnp.float32)
    w = a * jnp.cos(p)
    z = a * jnp.sin(p)
    out_ref[...] = (
        jnp.dot(w, sb_ref[...], preferred_element_type=jnp.float32)
        + jnp.dot(z, cb_ref[...], preferred_element_type=jnp.float32)
    )


def _tc_synth(ap_packed, sb, cb):
    grid = (_PR // _BR,)
    return pl.pallas_call(
        _tc_body,
        grid=grid,
        in_specs=[
            pl.BlockSpec((_BR, 128), lambda i: (i, 0)),
            pl.BlockSpec((128, 8 * EMBED_DIM), lambda i: (0, 0)),
            pl.BlockSpec((128, 8 * EMBED_DIM), lambda i: (0, 0)),
        ],
        out_specs=pl.BlockSpec((_BR, 8 * EMBED_DIM), lambda i: (i, 0)),
        out_shape=jax.ShapeDtypeStruct((_PR, 8 * EMBED_DIM), jnp.float32),
    )(ap_packed, sb, cb)


def kernel(x, frequency_amplitudes, frequency_phases, frequencies):
    idx = x.reshape(_T).astype(jnp.int32)
    base = idx + (idx // _W) * ((HARMONIC_BASES - 1) * _W)
    flat_ap = _flatten_pair(frequency_amplitudes, frequency_phases)
    ap_flat = _sc_gather(base, flat_ap)
    ap_packed = ap_flat.reshape(_PR, 128)

    t = jnp.linspace(0.0, 1.0, EMBED_DIM, dtype=jnp.float32)
    theta = (2.0 * math.pi) * frequencies[:, None] * t[None, :]
    eye8 = jnp.eye(8, dtype=jnp.float32)
    sb = jnp.kron(eye8, jnp.sin(theta))
    cb = jnp.kron(eye8, jnp.cos(theta))

    out = _tc_synth(ap_packed, sb, cb)
    return out.reshape(_B, _S, EMBED_DIM)
